# baseline (speedup 1.0000x reference)
import functools

import jax
import jax.numpy as jnp
from jax import lax
from jax.experimental import pallas as pl
from jax.experimental.pallas import tpu as pltpu
from jax.experimental.pallas import tpu_sc as plsc

NC = 2
NS = 16
NW = NC * NS
LANES = 16
IDXB = 128
CH_I = 8
CH_P = CH_I * IDXB
RW = 200


def _sc_partials(idx2d, inter, *, n_atoms, n_pairs):
    n_chunks = n_pairs // CH_P
    n_rchunks = n_atoms // RW

    mesh = plsc.VectorSubcoreMesh(core_axis_name="c", subcore_axis_name="s")

    @functools.partial(
        pl.kernel,
        out_type=jax.ShapeDtypeStruct((NC, n_atoms, LANES), jnp.float32),
        mesh=mesh,
        scratch_types=[
            pltpu.VMEM((CH_I, 1, IDXB), jnp.int32),
            pltpu.VMEM((CH_I, IDXB, LANES), jnp.float32),
            pltpu.VMEM((RW, LANES), jnp.float32),
            pltpu.VMEM_SHARED((n_atoms, LANES), jnp.float32),
            pltpu.SemaphoreType.DMA,
        ],
        compiler_params=pltpu.CompilerParams(use_tc_tiling_on_sc=False),
    )
    def body(idx_hbm, inter_hbm, out_hbm, idxv, rows, zbuf, acc, sem):
        c = lax.axis_index("c")
        s = lax.axis_index("s")
        w = s * NC + c

        def zero_row(i, _):
            zbuf[i] = jnp.zeros((LANES,), jnp.float32)
            return 0
        lax.fori_loop(0, RW, zero_row, 0)

        n_z = (n_rchunks - s + NS - 1) // NS

        def zero_chunk(z, _):
            zc = s + z * NS
            pltpu.sync_copy(zbuf, acc.at[pl.ds(zc * RW, RW)])
            return 0

        lax.fori_loop(0, n_z, zero_chunk, 0)
        plsc.subcore_barrier()

        n_k = (n_chunks - w + NW - 1) // NW

        lane = jnp.arange(LANES, dtype=jnp.int32)

        def chunk_body(k, _):
            cid = w + k * NW
            ld_p = pltpu.async_copy(
                idx_hbm.at[pl.ds(cid * CH_I, CH_I), pl.ds(0, 1)], idxv, sem)
            ld_r = pltpu.async_copy(
                inter_hbm.at[pl.ds(cid * CH_I, CH_I)], rows, sem)
            ld_p.wait()
            ld_r.wait()
            descs = [
                pltpu.async_copy(rows.at[j],
                                 acc.at[idxv.at[j, 0]], sem, add=True)
                for j in range(CH_I)
            ]
            for dsc in descs:
                dsc.wait()
            return 0

        lax.fori_loop(0, n_k, chunk_body, 0)
        plsc.subcore_barrier()

        def dump_chunk(z, _):
            zc = s + z * NS
            pltpu.sync_copy(acc.at[pl.ds(zc * RW, RW)],
                            out_hbm.at[c, pl.ds(zc * RW, RW)])
            return 0

        lax.fori_loop(0, n_z, dump_chunk, 0)

    return body(idx2d, inter)


def _merge_body(p_ref, o_ref):
    o_ref[...] = p_ref[0] + p_ref[1]


def _transpose_body(x_ref, y_ref):
    x = x_ref[...]
    tb = x.shape[1]
    x5 = x.reshape(2, tb, 8, 16, 8)
    y = x5.transpose(1, 3, 4, 0, 2).reshape(16 * tb, 128)
    y_ref[...] = y


def _tc_transpose(inter, *, n_pairs):
    nblk = n_pairs // 128
    tb = 125
    grid = nblk // tb
    assert nblk % tb == 0
    x4 = inter.reshape(nblk, 128, 2, 8).transpose(2, 0, 3, 1)
    return pl.pallas_call(
        _transpose_body,
        grid=(grid,),
        in_specs=[pl.BlockSpec((2, tb, 8, 128), lambda i: (0, i, 0, 0))],
        out_specs=pl.BlockSpec((16 * tb, 128), lambda i: (i, 0)),
        out_shape=jax.ShapeDtypeStruct((n_pairs // 8, 128), jnp.float32),
    )(x4)


def kernel(ind_2, prop, inter):
    n_atoms = prop.shape[0]
    n_pairs, n_inter = inter.shape
    assert n_inter == LANES
    assert n_pairs % CH_P == 0
    assert n_atoms % RW == 0
    assert (n_atoms * LANES) % 128 == 0

    idx3d = ind_2.reshape(n_pairs // IDXB, IDXB, 2).transpose(0, 2, 1)
    inter_w = jax.lax.optimization_barrier(inter.reshape(n_pairs // 8, IDXB))
    inter_p = inter_w.reshape(n_pairs // IDXB, IDXB, LANES)
    partials = _sc_partials(idx3d, inter_p, n_atoms=n_atoms, n_pairs=n_pairs)

    wide = n_atoms * LANES // 128
    pr = partials.reshape(NC, wide, 128)
    merged = pl.pallas_call(
        _merge_body,
        out_shape=jax.ShapeDtypeStruct((wide, 128), jnp.float32),
    )(pr)
    return merged.reshape(n_atoms, LANES)


# speedup vs baseline: 1.0227x; 1.0227x over previous
import functools

import jax
import jax.numpy as jnp
from jax import lax
from jax.experimental import pallas as pl
from jax.experimental.pallas import tpu as pltpu
from jax.experimental.pallas import tpu_sc as plsc

NC = 2
NS = 16
NW = NC * NS
LANES = 16
IDXB = 128
CH_I = 20
CH_P = CH_I * IDXB
RW = 200


def _sc_partials(idx2d, inter, *, n_atoms, n_pairs):
    n_chunks = n_pairs // CH_P
    n_rchunks = n_atoms // RW

    mesh = plsc.VectorSubcoreMesh(core_axis_name="c", subcore_axis_name="s")

    @functools.partial(
        pl.kernel,
        out_type=jax.ShapeDtypeStruct((NC, n_atoms, LANES), jnp.float32),
        mesh=mesh,
        scratch_types=[
            pltpu.VMEM((CH_I, 1, IDXB), jnp.int32),
            pltpu.VMEM((CH_I, IDXB, LANES), jnp.float32),
            pltpu.VMEM((RW, LANES), jnp.float32),
            pltpu.VMEM_SHARED((n_atoms, LANES), jnp.float32),
            pltpu.SemaphoreType.DMA,
        ],
        compiler_params=pltpu.CompilerParams(use_tc_tiling_on_sc=False),
    )
    def body(idx_hbm, inter_hbm, out_hbm, idxv, rows, zbuf, acc, sem):
        c = lax.axis_index("c")
        s = lax.axis_index("s")
        w = s * NC + c

        def zero_row(i, _):
            zbuf[i] = jnp.zeros((LANES,), jnp.float32)
            return 0
        lax.fori_loop(0, RW, zero_row, 0)

        n_z = (n_rchunks - s + NS - 1) // NS

        def zero_chunk(z, _):
            zc = s + z * NS
            pltpu.sync_copy(zbuf, acc.at[pl.ds(zc * RW, RW)])
            return 0

        lax.fori_loop(0, n_z, zero_chunk, 0)
        plsc.subcore_barrier()

        n_k = (n_chunks - w + NW - 1) // NW

        lane = jnp.arange(LANES, dtype=jnp.int32)

        def chunk_body(k, _):
            cid = w + k * NW
            ld_p = pltpu.async_copy(
                idx_hbm.at[pl.ds(cid * CH_I, CH_I), pl.ds(0, 1)], idxv, sem)
            ld_r = pltpu.async_copy(
                inter_hbm.at[pl.ds(cid * CH_I, CH_I)], rows, sem)
            ld_p.wait()
            ld_r.wait()
            descs = [
                pltpu.async_copy(rows.at[j],
                                 acc.at[idxv.at[j, 0]], sem, add=True)
                for j in range(CH_I)
            ]
            for dsc in descs:
                dsc.wait()
            return 0

        lax.fori_loop(0, n_k, chunk_body, 0)
        plsc.subcore_barrier()

        def dump_chunk(z, _):
            zc = s + z * NS
            pltpu.sync_copy(acc.at[pl.ds(zc * RW, RW)],
                            out_hbm.at[c, pl.ds(zc * RW, RW)])
            return 0

        lax.fori_loop(0, n_z, dump_chunk, 0)

    return body(idx2d, inter)


def _merge_body(p_ref, o_ref):
    o_ref[...] = p_ref[0] + p_ref[1]


def _transpose_body(x_ref, y_ref):
    x = x_ref[...]
    tb = x.shape[1]
    x5 = x.reshape(2, tb, 8, 16, 8)
    y = x5.transpose(1, 3, 4, 0, 2).reshape(16 * tb, 128)
    y_ref[...] = y


def _tc_transpose(inter, *, n_pairs):
    nblk = n_pairs // 128
    tb = 125
    grid = nblk // tb
    assert nblk % tb == 0
    x4 = inter.reshape(nblk, 128, 2, 8).transpose(2, 0, 3, 1)
    return pl.pallas_call(
        _transpose_body,
        grid=(grid,),
        in_specs=[pl.BlockSpec((2, tb, 8, 128), lambda i: (0, i, 0, 0))],
        out_specs=pl.BlockSpec((16 * tb, 128), lambda i: (i, 0)),
        out_shape=jax.ShapeDtypeStruct((n_pairs // 8, 128), jnp.float32),
    )(x4)


def kernel(ind_2, prop, inter):
    n_atoms = prop.shape[0]
    n_pairs, n_inter = inter.shape
    assert n_inter == LANES
    assert n_pairs % CH_P == 0
    assert n_atoms % RW == 0
    assert (n_atoms * LANES) % 128 == 0

    idx3d = ind_2.reshape(n_pairs // IDXB, IDXB, 2).transpose(0, 2, 1)
    inter_p = inter.reshape(n_pairs // IDXB, IDXB, LANES)
    partials = _sc_partials(idx3d, inter_p, n_atoms=n_atoms, n_pairs=n_pairs)

    wide = n_atoms * LANES // 128
    pr = partials.reshape(NC, wide, 128)
    merged = pl.pallas_call(
        _merge_body,
        out_shape=jax.ShapeDtypeStruct((wide, 128), jnp.float32),
    )(pr)
    return merged.reshape(n_atoms, LANES)


# speedup vs baseline: 1.0603x; 1.0367x over previous
import functools

import jax
import jax.numpy as jnp
from jax import lax
from jax.experimental import pallas as pl
from jax.experimental.pallas import tpu as pltpu
from jax.experimental.pallas import tpu_sc as plsc

NC = 2
NS = 16
NW = NC * NS
LANES = 16
IDXB = 128
CH_I = 10
CH_P = CH_I * IDXB
RW = 200


def _sc_partials(idx2d, inter, *, n_atoms, n_pairs):
    n_chunks = n_pairs // CH_P
    n_rchunks = n_atoms // RW

    mesh = plsc.VectorSubcoreMesh(core_axis_name="c", subcore_axis_name="s")

    @functools.partial(
        pl.kernel,
        out_type=jax.ShapeDtypeStruct((NC, n_atoms, LANES), jnp.float32),
        mesh=mesh,
        scratch_types=[
            pltpu.VMEM((CH_I, 1, IDXB), jnp.int32),
            pltpu.VMEM((CH_I, 1, IDXB), jnp.int32),
            pltpu.VMEM((CH_I, IDXB, LANES), jnp.float32),
            pltpu.VMEM((CH_I, IDXB, LANES), jnp.float32),
            pltpu.VMEM((RW, LANES), jnp.float32),
            pltpu.VMEM_SHARED((n_atoms, LANES), jnp.float32),
            pltpu.SemaphoreType.DMA,
            pltpu.SemaphoreType.DMA,
            pltpu.SemaphoreType.DMA,
        ],
        compiler_params=pltpu.CompilerParams(use_tc_tiling_on_sc=False),
    )
    def body(idx_hbm, inter_hbm, out_hbm, iv0, iv1, rw0, rw1, zbuf, acc,
             lsem0, lsem1, ssem):
        c = lax.axis_index("c")
        s = lax.axis_index("s")
        w = s * NC + c

        def zero_row(i, _):
            zbuf[i] = jnp.zeros((LANES,), jnp.float32)
            return 0
        lax.fori_loop(0, RW, zero_row, 0)

        n_z = (n_rchunks - s + NS - 1) // NS

        def zero_chunk(z, _):
            zc = s + z * NS
            pltpu.sync_copy(zbuf, acc.at[pl.ds(zc * RW, RW)])
            return 0

        lax.fori_loop(0, n_z, zero_chunk, 0)
        plsc.subcore_barrier()

        n_k = (n_chunks - w + NW - 1) // NW

        def start(k, iv, rw, sem):
            cid = (w + jnp.minimum(k, n_k - 1) * NW) * CH_I
            pltpu.async_copy(idx_hbm.at[pl.ds(cid, CH_I), pl.ds(0, 1)],
                             iv, sem)
            pltpu.async_copy(inter_hbm.at[pl.ds(cid, CH_I)], rw, sem)

        def wait_l(iv, rw, sem):
            pltpu.make_async_copy(
                idx_hbm.at[pl.ds(0, CH_I), pl.ds(0, 1)], iv, sem).wait()
            pltpu.make_async_copy(
                inter_hbm.at[pl.ds(0, CH_I)], rw, sem).wait()

        def process(iv, rw):
            descs = [
                pltpu.async_copy(rw.at[j], acc.at[iv.at[j, 0]], ssem,
                                 add=True)
                for j in range(CH_I)
            ]
            for dsc in descs:
                dsc.wait()

        start(0, iv0, rw0, lsem0)

        def duo(d, _):
            start(2 * d + 1, iv1, rw1, lsem1)
            wait_l(iv0, rw0, lsem0)
            process(iv0, rw0)
            start(2 * d + 2, iv0, rw0, lsem0)
            wait_l(iv1, rw1, lsem1)

            @pl.when(2 * d + 1 < n_k)
            def _odd():
                process(iv1, rw1)
            return 0

        lax.fori_loop(0, (n_k + 1) // 2, duo, 0)
        wait_l(iv0, rw0, lsem0)
        plsc.subcore_barrier()

        def dump_chunk(z, _):
            zc = s + z * NS
            pltpu.sync_copy(acc.at[pl.ds(zc * RW, RW)],
                            out_hbm.at[c, pl.ds(zc * RW, RW)])
            return 0

        lax.fori_loop(0, n_z, dump_chunk, 0)

    return body(idx2d, inter)


def _merge_body(p_ref, o_ref):
    o_ref[...] = p_ref[0] + p_ref[1]


def _transpose_body(x_ref, y_ref):
    x = x_ref[...]
    tb = x.shape[1]
    x5 = x.reshape(2, tb, 8, 16, 8)
    y = x5.transpose(1, 3, 4, 0, 2).reshape(16 * tb, 128)
    y_ref[...] = y


def _tc_transpose(inter, *, n_pairs):
    nblk = n_pairs // 128
    tb = 125
    grid = nblk // tb
    assert nblk % tb == 0
    x4 = inter.reshape(nblk, 128, 2, 8).transpose(2, 0, 3, 1)
    return pl.pallas_call(
        _transpose_body,
        grid=(grid,),
        in_specs=[pl.BlockSpec((2, tb, 8, 128), lambda i: (0, i, 0, 0))],
        out_specs=pl.BlockSpec((16 * tb, 128), lambda i: (i, 0)),
        out_shape=jax.ShapeDtypeStruct((n_pairs // 8, 128), jnp.float32),
    )(x4)


def kernel(ind_2, prop, inter):
    n_atoms = prop.shape[0]
    n_pairs, n_inter = inter.shape
    assert n_inter == LANES
    assert n_pairs % CH_P == 0
    assert n_atoms % RW == 0
    assert (n_atoms * LANES) % 128 == 0

    idx3d = ind_2.reshape(n_pairs // IDXB, IDXB, 2).transpose(0, 2, 1)
    inter_p = inter.reshape(n_pairs // IDXB, IDXB, LANES)
    partials = _sc_partials(idx3d, inter_p, n_atoms=n_atoms, n_pairs=n_pairs)

    wide = n_atoms * LANES // 128
    pr = partials.reshape(NC, wide, 128)
    merged = pl.pallas_call(
        _merge_body,
        out_shape=jax.ShapeDtypeStruct((wide, 128), jnp.float32),
    )(pr)
    return merged.reshape(n_atoms, LANES)


# speedup vs baseline: 1.0609x; 1.0005x over previous
import functools

import jax
import jax.numpy as jnp
from jax import lax
from jax.experimental import pallas as pl
from jax.experimental.pallas import tpu as pltpu
from jax.experimental.pallas import tpu_sc as plsc

NC = 2
NS = 16
NW = NC * NS
LANES = 16
IDXB = 128
CH_I = 10
CH_P = CH_I * IDXB
RW = 200


def _sc_partials(idx2d, inter, *, n_atoms, n_pairs):
    n_chunks = n_pairs // CH_P
    n_rchunks = n_atoms // RW

    mesh = plsc.VectorSubcoreMesh(core_axis_name="c", subcore_axis_name="s")

    @functools.partial(
        pl.kernel,
        out_type=jax.ShapeDtypeStruct((NC, n_atoms, LANES), jnp.float32),
        mesh=mesh,
        scratch_types=[
            pltpu.VMEM((CH_I, 1, IDXB), jnp.int32),
            pltpu.VMEM((CH_I, 1, IDXB), jnp.int32),
            pltpu.VMEM((CH_I, IDXB, LANES), jnp.float32),
            pltpu.VMEM((CH_I, IDXB, LANES), jnp.float32),
            pltpu.VMEM((RW, LANES), jnp.float32),
            pltpu.VMEM_SHARED((n_atoms, LANES), jnp.float32),
            pltpu.SemaphoreType.DMA,
            pltpu.SemaphoreType.DMA,
            pltpu.SemaphoreType.DMA,
        ],
        compiler_params=pltpu.CompilerParams(use_tc_tiling_on_sc=False),
    )
    def body(idx_hbm, inter_hbm, out_hbm, iv0, iv1, rw0, rw1, zbuf, acc,
             lsem0, lsem1, ssem):
        c = lax.axis_index("c")
        s = lax.axis_index("s")
        w = s * NC + c

        def zero_row(i, _):
            zbuf[i] = jnp.zeros((LANES,), jnp.float32)
            return 0
        lax.fori_loop(0, RW, zero_row, 0)

        n_z = (n_rchunks - s + NS - 1) // NS

        def zero_chunk(z, _):
            zc = s + z * NS
            pltpu.sync_copy(zbuf, acc.at[pl.ds(zc * RW, RW)])
            return 0

        lax.fori_loop(0, n_z, zero_chunk, 0)
        plsc.subcore_barrier()

        n_k = (n_chunks - w + NW - 1) // NW

        def start(k, iv, rw, sem):
            cid = (w + jnp.minimum(k, n_k - 1) * NW) * CH_I
            pltpu.async_copy(idx_hbm.at[pl.ds(cid, CH_I), pl.ds(0, 1)],
                             iv, sem)
            pltpu.async_copy(inter_hbm.at[pl.ds(cid, CH_I)], rw, sem)

        def wait_l(iv, rw, sem):
            pltpu.make_async_copy(
                idx_hbm.at[pl.ds(0, CH_I), pl.ds(0, 1)], iv, sem).wait()
            pltpu.make_async_copy(
                inter_hbm.at[pl.ds(0, CH_I)], rw, sem).wait()

        def process(iv, rw):
            descs = [
                pltpu.async_copy(rw.at[j], acc.at[iv.at[j, 0]], ssem,
                                 add=True)
                for j in range(CH_I)
            ]
            for dsc in descs:
                dsc.wait()

        start(0, iv0, rw0, lsem0)

        def duo(d, _):
            start(2 * d + 1, iv1, rw1, lsem1)
            wait_l(iv0, rw0, lsem0)
            process(iv0, rw0)
            start(2 * d + 2, iv0, rw0, lsem0)
            wait_l(iv1, rw1, lsem1)

            @pl.when(2 * d + 1 < n_k)
            def _odd():
                process(iv1, rw1)
            return 0

        lax.fori_loop(0, (n_k + 1) // 2, duo, 0)
        wait_l(iv0, rw0, lsem0)
        plsc.subcore_barrier()

        def dump_chunk(z, _):
            zc = s + z * NS
            pltpu.sync_copy(acc.at[pl.ds(zc * RW, RW)],
                            out_hbm.at[c, pl.ds(zc * RW, RW)])
            return 0

        lax.fori_loop(0, n_z, dump_chunk, 0)

    return body(idx2d, inter)


def _merge_body(p_ref, o_ref):
    o_ref[...] = p_ref[0] + p_ref[1]


def kernel(ind_2, prop, inter):
    n_atoms = prop.shape[0]
    n_pairs, n_inter = inter.shape
    assert n_inter == LANES
    assert n_pairs % CH_P == 0
    assert n_atoms % RW == 0
    assert (n_atoms * LANES) % 128 == 0

    idx3d = ind_2.reshape(n_pairs // IDXB, IDXB, 2).transpose(0, 2, 1)
    inter_p = inter.reshape(n_pairs // IDXB, IDXB, LANES)
    partials = _sc_partials(idx3d, inter_p, n_atoms=n_atoms, n_pairs=n_pairs)

    wide = n_atoms * LANES // 128
    pr = partials.reshape(NC, wide, 128)
    merged = pl.pallas_call(
        _merge_body,
        out_shape=jax.ShapeDtypeStruct((wide, 128), jnp.float32),
    )(pr)
    return merged.reshape(n_atoms, LANES)
